# Initial kernel scaffold; baseline (speedup 1.0000x reference)
import jax
import jax.numpy as jnp
from jax.experimental import pallas as pl


def kernel(indices, codebook, W, b):
    raise NotImplementedError("write your pallas kernel here")


# speedup vs baseline: 2.2133x; 2.2133x over previous
import functools

import jax
import jax.numpy as jnp
from jax import lax
from jax.experimental import pallas as pl
from jax.experimental.pallas import tpu as pltpu
from jax.experimental.pallas import tpu_sc as plsc

_B, _T, _K, _CD, _DIM = 8, 4096, 8192, 256, 1024
_N = _B * _T
_NW = 32
_BPW = _N // _NW
_CHUNK = 128
_NCHUNK = _BPW // _CHUNK
_TT = 512


def _sc_gather_body(table_hbm, idx_hbm, out_hbm, idx_v, rows0, rows1, sem0, sem1):
    wid = lax.axis_index("s") * 2 + lax.axis_index("c")
    base = wid * _BPW
    pltpu.sync_copy(idx_hbm.at[wid], idx_v)
    bufs = (rows0, rows1)
    sems = (sem0, sem1)
    cp = pltpu.async_copy(table_hbm.at[idx_v.at[0]], bufs[0], sems[0])
    for c in range(_NCHUNK):
        cp.wait()
        if c + 1 < _NCHUNK:
            cp = pltpu.async_copy(
                table_hbm.at[idx_v.at[c + 1]], bufs[(c + 1) % 2], sems[(c + 1) % 2]
            )
        pltpu.sync_copy(bufs[c % 2], out_hbm.at[pl.ds(base + c * _CHUNK, _CHUNK)])


_sc_gather = functools.partial(
    pl.kernel,
    mesh=plsc.VectorSubcoreMesh(core_axis_name="c", subcore_axis_name="s"),
    out_type=jax.ShapeDtypeStruct((_N, _CD), jnp.float32),
    scratch_types=[
        pltpu.VMEM((_NCHUNK, _CHUNK), jnp.int32),
        pltpu.VMEM((_CHUNK, _CD), jnp.float32),
        pltpu.VMEM((_CHUNK, _CD), jnp.float32),
        pltpu.SemaphoreType.DMA,
        pltpu.SemaphoreType.DMA,
    ],
)(_sc_gather_body)


def _mm_body(x_ref, w_ref, b_ref, o_ref):
    x = x_ref[0]
    w = w_ref[...]
    acc = lax.dot_general(
        w, x, (((1,), (1,)), ((), ())), preferred_element_type=jnp.float32
    )
    o_ref[0] = acc + b_ref[...]


def kernel(indices, codebook, W, b):
    idx = indices.reshape(_NW, _NCHUNK, _CHUNK).astype(jnp.int32)
    gathered = _sc_gather(codebook, idx)
    out = pl.pallas_call(
        _mm_body,
        grid=(_B, _T // _TT),
        in_specs=[
            pl.BlockSpec((1, _TT, _CD), lambda bb, tt: (bb * (_T // _TT) + tt, 0, 0)),
            pl.BlockSpec((_DIM, _CD), lambda bb, tt: (0, 0)),
            pl.BlockSpec((_DIM, 1), lambda bb, tt: (0, 0)),
        ],
        out_specs=pl.BlockSpec((1, _DIM, _TT), lambda bb, tt: (bb, 0, tt)),
        out_shape=jax.ShapeDtypeStruct((_B, _DIM, _T), jnp.float32),
    )(gathered.reshape(_N // _TT, _TT, _CD), W, b.reshape(_DIM, 1))
    return out
